# Initial kernel scaffold; baseline (speedup 1.0000x reference)
import jax
import jax.numpy as jnp
from jax.experimental import pallas as pl


def kernel(x, edge_index, batch, pre_W1, pre_b1, bn_g, bn_b, pre_W2, pre_b2, mid_W1, mid_b1, mid_W2, mid_b2, post_W1, post_b1, post_W2, post_b2, f_W1, f_b1, f_W2, f_b2, f_W3, f_b3):
    raise NotImplementedError("write your pallas kernel here")


# speedup vs baseline: 6.4858x; 6.4858x over previous
import functools

import jax
import jax.numpy as jnp
from jax import lax
from jax.experimental import pallas as pl
from jax.experimental.pallas import tpu as pltpu
from jax.experimental.pallas import tpu_sc as plsc

N = 10000
E = 320000
D_IN = 128
H = 64
OUT = 10
G = 64

CH = 128
NCHUNK = E // CH
NTILES = 32
N_PAD = 10240
ROWS_PER_TILE = N_PAD // 16


def _elu(v):
    return jnp.where(v > 0, v, jnp.exp(jnp.minimum(v, 0.0)) - 1.0)


def _sc_segsum(y, src2, dst2, zrows):
    mesh = plsc.VectorSubcoreMesh(core_axis_name="c", subcore_axis_name="s")

    @functools.partial(
        pl.kernel,
        out_type=jax.ShapeDtypeStruct((2, N_PAD, H), jnp.float32),
        mesh=mesh,
        scratch_types=[
            pltpu.VMEM((CH,), jnp.int32),
            pltpu.VMEM((CH,), jnp.int32),
            pltpu.VMEM((CH, H), jnp.float32),
            pltpu.VMEM_SHARED((N_PAD, H), jnp.float32),
            pltpu.SemaphoreType.DMA,
        ],
        compiler_params=pltpu.CompilerParams(use_tc_tiling_on_sc=False),
    )
    def k(y_hbm, src_hbm, dst_hbm, z_hbm, out_hbm, src_v, dst_v, rows_v,
          acc_sh, sem):
        cid = lax.axis_index("c")
        sid = lax.axis_index("s")
        wid = sid * 2 + cid
        row0 = sid * ROWS_PER_TILE
        pltpu.sync_copy(z_hbm, acc_sh.at[pl.ds(row0, ROWS_PER_TILE)])
        plsc.subcore_barrier()

        @pl.loop(0, (NCHUNK + NTILES - 1) // NTILES)
        def _(i):
            c = i * NTILES + wid

            @pl.when(c < NCHUNK)
            def _():
                pltpu.sync_copy(src_hbm.at[c], src_v)
                pltpu.sync_copy(dst_hbm.at[c], dst_v)
                pltpu.async_copy(y_hbm.at[src_v], rows_v, sem).wait()
                pltpu.sync_copy(rows_v, acc_sh.at[dst_v], add=True)

        plsc.subcore_barrier()
        pltpu.sync_copy(acc_sh.at[pl.ds(row0, ROWS_PER_TILE)],
                        out_hbm.at[cid, pl.ds(row0, ROWS_PER_TILE)])

    return k(y, src2, dst2, zrows)


def _tc(body, out_shape, *args):
    return pl.pallas_call(
        body, out_shape=jax.ShapeDtypeStruct(out_shape, jnp.float32))(*args)


def _proj_body(x_ref, w_ref, o_ref):
    o_ref[...] = jnp.dot(x_ref[...], w_ref[...],
                         preferred_element_type=jnp.float32)


def _pre_body(y_ref, p_ref, b1_ref, g_ref, bb_ref, w2_ref, b2_ref, o_ref):
    h = y_ref[...] + p_ref[0, :N] + p_ref[1, :N] + b1_ref[...]
    mu = jnp.mean(h, axis=0, keepdims=True)
    var = jnp.mean((h - mu) * (h - mu), axis=0, keepdims=True)
    h = (h - mu) * lax.rsqrt(var + 1e-5) * g_ref[...] + bb_ref[...]
    h = _elu(h)
    h = jnp.dot(h, w2_ref[...], preferred_element_type=jnp.float32) + b2_ref[...]
    o_ref[...] = _elu(h)


def _mid_body(h_ref, p_ref, w1_ref, b1_ref, w2_ref, b2_ref, o_ref):
    h = h_ref[...] + p_ref[0, :N] + p_ref[1, :N]
    h = _elu(jnp.dot(h, w1_ref[...], preferred_element_type=jnp.float32)
             + b1_ref[...])
    h = jnp.dot(h, w2_ref[...], preferred_element_type=jnp.float32) + b2_ref[...]
    o_ref[...] = _elu(h)


def _post_body(h_ref, p_ref, batch_ref, w1_ref, b1_ref, w2_ref, b2_ref,
               fw1_ref, fb1_ref, fw2_ref, fb2_ref, fw3_ref, fb3_ref, o_ref):
    h = h_ref[...] + p_ref[0, :N] + p_ref[1, :N]
    h = _elu(jnp.dot(h, w1_ref[...], preferred_element_type=jnp.float32)
             + b1_ref[...])
    h = jnp.dot(h, w2_ref[...], preferred_element_type=jnp.float32) + b2_ref[...]
    h = _elu(h)
    gids = lax.broadcasted_iota(jnp.int32, (G, N), 0)
    mask = (gids == batch_ref[...]).astype(jnp.float32)
    sums = jnp.dot(mask, h, preferred_element_type=jnp.float32)
    cnt = jnp.sum(mask, axis=1, keepdims=True)
    pooled = sums / jnp.maximum(cnt, 1.0)
    z = _elu(jnp.dot(pooled, fw1_ref[...], preferred_element_type=jnp.float32)
             + fb1_ref[...])
    z = _elu(jnp.dot(z, fw2_ref[...], preferred_element_type=jnp.float32)
             + fb2_ref[...])
    o_ref[...] = jnp.dot(z, fw3_ref[...],
                         preferred_element_type=jnp.float32) + fb3_ref[...]


def kernel(x, edge_index, batch, pre_W1, pre_b1, bn_g, bn_b, pre_W2, pre_b2,
           mid_W1, mid_b1, mid_W2, mid_b2, post_W1, post_b1, post_W2, post_b2,
           f_W1, f_b1, f_W2, f_b2, f_W3, f_b3):
    src2 = edge_index[0].reshape(NCHUNK, CH)
    dst2 = edge_index[1].reshape(NCHUNK, CH)
    zrows = jnp.zeros((ROWS_PER_TILE, H), jnp.float32)
    batch_row = batch.reshape(1, N)
    r1 = lambda v: v.reshape(1, -1)

    y = _tc(_proj_body, (N, H), x, pre_W1)
    p = _sc_segsum(y, src2, dst2, zrows)
    h1 = _tc(_pre_body, (N, H), y, p, r1(pre_b1), r1(bn_g), r1(bn_b),
             pre_W2, r1(pre_b2))
    q = _sc_segsum(h1, src2, dst2, zrows)
    h2 = _tc(_mid_body, (N, H), h1, q, mid_W1, r1(mid_b1), mid_W2, r1(mid_b2))
    r = _sc_segsum(h2, src2, dst2, zrows)
    out = _tc(_post_body, (G, OUT), h2, r, batch_row,
              post_W1, r1(post_b1), post_W2, r1(post_b2),
              f_W1, r1(f_b1), f_W2, r1(f_b2), f_W3, r1(f_b3))
    return out
